# Initial kernel scaffold; baseline (speedup 1.0000x reference)
import jax
import jax.numpy as jnp
from jax.experimental import pallas as pl


def kernel(x, edge_index, batch, W1, a_s1, a_d1, b1, W2, a_s2, a_d2, b2, W3, a_s3, a_d3, b3):
    raise NotImplementedError("write your pallas kernel here")


# speedup vs baseline: 17.0715x; 17.0715x over previous
import functools

import jax
import jax.numpy as jnp
from jax import lax
from jax.experimental import pallas as pl
from jax.experimental.pallas import tpu as pltpu
from jax.experimental.pallas import tpu_sc as plsc

N = 10000
E = 320000
F_IN = 128
HID = 64
H = 4
G = 128

NC = 2
NS = 16
L = 16

K = 80
BLK = 1000
NBLK = N // BLK
ROWS_PER_TILE = 624

_mesh = plsc.VectorSubcoreMesh(
    core_axis_name="c", subcore_axis_name="s", num_cores=NC, num_subcores=NS)


def _zero_rows(ref, nrows, ncols):
    def body(r, _):
        for q in range(ncols // L):
            ref[r, pl.ds(q * L, L)] = jnp.zeros((L,), jnp.float32)
        return 0
    lax.fori_loop(0, nrows, body, 0)


_ROW_CHUNKS = [K] * 7 + [ROWS_PER_TILE - 7 * K]


def _zero_acc(acc, stage, s):
    base = s * ROWS_PER_TILE
    off = 0
    for ln in _ROW_CHUNKS:
        pltpu.sync_copy(stage.at[pl.ds(0, ln)], acc.at[pl.ds(base + off, ln)])
        off += ln

    @pl.when(s == NS - 1)
    def _():
        pltpu.sync_copy(stage.at[pl.ds(0, 16)],
                        acc.at[pl.ds(NS * ROWS_PER_TILE, 16)])


def _acc_writeback(acc, stage, out, c, s):
    base = s * ROWS_PER_TILE
    off = 0
    for ln in _ROW_CHUNKS:
        pltpu.sync_copy(acc.at[pl.ds(base + off, ln)], stage.at[pl.ds(0, ln)])
        pltpu.sync_copy(stage.at[pl.ds(0, ln)],
                        out.at[c, pl.ds(base + off, ln)])
        off += ln

    @pl.when(s == NS - 1)
    def _():
        r0 = NS * ROWS_PER_TILE
        pltpu.sync_copy(acc.at[pl.ds(r0, 16)], stage.at[pl.ds(0, 16)])
        pltpu.sync_copy(stage.at[pl.ds(0, 16)], out.at[c, pl.ds(r0, 16)])


@functools.partial(
    pl.kernel,
    out_type=jax.ShapeDtypeStruct((NC, N, 144), jnp.float32),
    mesh=_mesh,
    compiler_params=pltpu.CompilerParams(use_tc_tiling_on_sc=False),
    scratch_types=[
        pltpu.VMEM_SHARED((N, 144), jnp.float32),
        pltpu.VMEM((K,), jnp.int32),
        pltpu.VMEM((K,), jnp.int32),
        pltpu.VMEM((K,), jnp.int32),
        pltpu.VMEM((K,), jnp.int32),
        pltpu.VMEM((K, 16), jnp.float32),
        pltpu.VMEM((K, 16), jnp.float32),
        pltpu.VMEM((K, 128), jnp.float32),
        pltpu.VMEM((K, 144), jnp.float32),
        pltpu.SemaphoreType.DMA,
        pltpu.SemaphoreType.DMA,
        pltpu.SemaphoreType.DMA,
    ],
)
def _sc_edges4(srce, dste, hsplit, asel, dsel, out,
               acc, src_b, dst_b, adjs_b, adjd_b, as_r, ad_r, h_r, stage,
               sem0, sem1, sem2):
    c = lax.axis_index("c")
    s = lax.axis_index("s")
    riota = lax.iota(jnp.int32, L)

    _zero_rows(stage, K, 144)
    _zero_acc(acc, stage, s)
    plsc.subcore_barrier()

    epw = E // NS

    def chunk(i, _):
        base = s * epw + i * K
        pltpu.sync_copy(srce.at[pl.ds(base, K)], src_b)
        pltpu.sync_copy(dste.at[pl.ds(base, K)], dst_b)
        for q in range(K // L):
            adjs_b[pl.ds(q * L, L)] = src_b[pl.ds(q * L, L)] + c * N
            adjd_b[pl.ds(q * L, L)] = dst_b[pl.ds(q * L, L)] + c * N
        cph = pltpu.async_copy(hsplit.at[adjs_b], h_r, sem0)
        cpa = pltpu.async_copy(asel.at[adjs_b], as_r, sem1)
        cpb = pltpu.async_copy(dsel.at[adjd_b], ad_r, sem2)
        cpa.wait()
        cpb.wait()
        cph.wait()

        def scale(e, _):
            t = as_r[e, :] + ad_r[e, :]
            w = jnp.exp(jnp.where(t >= 0, t, 0.2 * t))
            w0 = w[0]
            w1 = w[1]
            for q in range(4):
                stage[e, pl.ds(q * L, L)] = h_r[e, pl.ds(q * L, L)] * w0
            for q in range(4):
                stage[e, pl.ds(64 + q * L, L)] = h_r[e, pl.ds(64 + q * L, L)] * w1
            stage[e, pl.ds(128, L)] = jnp.where(riota < 2, w, 0.0)
            return 0
        lax.fori_loop(0, K, scale, 0)
        pltpu.sync_copy(stage, acc.at[dst_b], add=True)
        return 0

    lax.fori_loop(0, epw // K, chunk, 0)
    plsc.subcore_barrier()
    _acc_writeback(acc, stage, out, c, s)


@functools.partial(
    pl.kernel,
    out_type=jax.ShapeDtypeStruct((NC, N, 80), jnp.float32),
    mesh=_mesh,
    compiler_params=pltpu.CompilerParams(use_tc_tiling_on_sc=False),
    scratch_types=[
        pltpu.VMEM_SHARED((N, 80), jnp.float32),
        pltpu.VMEM((K,), jnp.int32),
        pltpu.VMEM((K,), jnp.int32),
        pltpu.VMEM((K, 16), jnp.float32),
        pltpu.VMEM((K, 16), jnp.float32),
        pltpu.VMEM((K, 64), jnp.float32),
        pltpu.VMEM((K, 80), jnp.float32),
        pltpu.SemaphoreType.DMA,
        pltpu.SemaphoreType.DMA,
        pltpu.SemaphoreType.DMA,
    ],
)
def _sc_edges1(srce, dste, h3, asrc, adst, out,
               acc, src_b, dst_b, as_r, ad_r, h_r, stage,
               sem0, sem1, sem2):
    c = lax.axis_index("c")
    s = lax.axis_index("s")
    riota = lax.iota(jnp.int32, L)

    _zero_rows(stage, K, 80)
    _zero_acc(acc, stage, s)
    plsc.subcore_barrier()

    epw = E // (NC * NS)

    def chunk(i, _):
        base = (c * NS + s) * epw + i * K
        pltpu.sync_copy(srce.at[pl.ds(base, K)], src_b)
        pltpu.sync_copy(dste.at[pl.ds(base, K)], dst_b)
        cph = pltpu.async_copy(h3.at[src_b], h_r, sem0)
        cpa = pltpu.async_copy(asrc.at[src_b], as_r, sem1)
        cpb = pltpu.async_copy(adst.at[dst_b], ad_r, sem2)
        cpa.wait()
        cpb.wait()
        cph.wait()

        def scale(e, _):
            t = as_r[e, :] + ad_r[e, :]
            w = jnp.exp(jnp.where(t >= 0, t, 0.2 * t))
            w0 = w[0]
            for q in range(4):
                stage[e, pl.ds(q * L, L)] = h_r[e, pl.ds(q * L, L)] * w0
            stage[e, pl.ds(64, L)] = jnp.where(riota < 1, w, 0.0)
            return 0
        lax.fori_loop(0, K, scale, 0)
        pltpu.sync_copy(stage, acc.at[dst_b], add=True)
        return 0

    lax.fori_loop(0, epw // K, chunk, 0)
    plsc.subcore_barrier()
    _acc_writeback(acc, stage, out, c, s)


def _emit_scores(ap_ref, ap):
    ap_ref[0] = ap
    ap_ref[1] = jnp.concatenate(
        [ap[:, 2:], jnp.zeros((ap.shape[0], 2), jnp.float32)], axis=1)


def _tc1_body(x_ref, w_ref, as_ref, ad_ref, hs_ref, ap_ref, dp_ref):
    h = jnp.dot(x_ref[...], w_ref[...], preferred_element_type=jnp.float32)
    hs_ref[0] = h[:, :128]
    hs_ref[1] = h[:, 128:]
    _emit_scores(ap_ref, jnp.dot(h, as_ref[...],
                                 preferred_element_type=jnp.float32))
    _emit_scores(dp_ref, jnp.dot(h, ad_ref[...],
                                 preferred_element_type=jnp.float32))


def _tc1(x, W1, As, Ad):
    return pl.pallas_call(
        _tc1_body,
        grid=(NBLK,),
        in_specs=[
            pl.BlockSpec((BLK, F_IN), lambda i: (i, 0)),
            pl.BlockSpec((F_IN, H * HID), lambda i: (0, 0)),
            pl.BlockSpec((H * HID, 16), lambda i: (0, 0)),
            pl.BlockSpec((H * HID, 16), lambda i: (0, 0)),
        ],
        out_specs=[
            pl.BlockSpec((NC, BLK, 128), lambda i: (0, i, 0)),
            pl.BlockSpec((NC, BLK, 16), lambda i: (0, i, 0)),
            pl.BlockSpec((NC, BLK, 16), lambda i: (0, i, 0)),
        ],
        out_shape=[
            jax.ShapeDtypeStruct((NC, N, 128), jnp.float32),
            jax.ShapeDtypeStruct((NC, N, 16), jnp.float32),
            jax.ShapeDtypeStruct((NC, N, 16), jnp.float32),
        ],
    )(x, W1, As, Ad)


def _combine4(acc_ref, hs_ref, ap_ref, dp_ref, b_ref):
    t = ap_ref[0] + dp_ref[0]
    ws = jnp.exp(jnp.where(t >= 0, t, 0.2 * t))
    cols = []
    for hh in range(H):
        scid, j = hh // 2, hh % 2
        w = ws[:, hh:hh + 1]
        num = acc_ref[scid][:, j * 64:(j + 1) * 64] \
            + w * hs_ref[scid][:, j * 64:(j + 1) * 64]
        den = acc_ref[scid][:, 128 + j:129 + j] + w
        cols.append(num / den)
    return jnp.concatenate(cols, axis=1) + b_ref[...]


def _tc2_body(acc_ref, hs_ref, ap_ref, dp_ref, b_ref, w_ref, as_ref, ad_ref,
              hs2_ref, ap2_ref, dp2_ref):
    o = _combine4(acc_ref, hs_ref, ap_ref, dp_ref, b_ref)
    x2 = jnp.where(o > 0, o, jnp.exp(o) - 1.0)
    h2 = jnp.dot(x2, w_ref[...], preferred_element_type=jnp.float32)
    if h2.shape[1] == H * HID:
        hs2_ref[0] = h2[:, :128]
        hs2_ref[1] = h2[:, 128:]
    else:
        hs2_ref[...] = h2
    _emit_scores(ap2_ref, jnp.dot(h2, as_ref[...],
                                  preferred_element_type=jnp.float32))
    _emit_scores(dp2_ref, jnp.dot(h2, ad_ref[...],
                                  preferred_element_type=jnp.float32))


def _tc_mid(acc, hs, ap, dp, b, W, As, Ad, out_heads):
    wide = out_heads * HID
    if out_heads == H:
        out0_spec = pl.BlockSpec((NC, BLK, 128), lambda i: (0, i, 0))
        out0_shape = jax.ShapeDtypeStruct((NC, N, 128), jnp.float32)
    else:
        out0_spec = pl.BlockSpec((BLK, HID), lambda i: (i, 0))
        out0_shape = jax.ShapeDtypeStruct((N, HID), jnp.float32)
    return pl.pallas_call(
        _tc2_body,
        grid=(NBLK,),
        in_specs=[
            pl.BlockSpec((NC, BLK, 144), lambda i: (0, i, 0)),
            pl.BlockSpec((NC, BLK, 128), lambda i: (0, i, 0)),
            pl.BlockSpec((NC, BLK, 16), lambda i: (0, i, 0)),
            pl.BlockSpec((NC, BLK, 16), lambda i: (0, i, 0)),
            pl.BlockSpec((1, H * HID), lambda i: (0, 0)),
            pl.BlockSpec((H * HID, wide), lambda i: (0, 0)),
            pl.BlockSpec((wide, 16), lambda i: (0, 0)),
            pl.BlockSpec((wide, 16), lambda i: (0, 0)),
        ],
        out_specs=[
            out0_spec,
            pl.BlockSpec((NC, BLK, 16), lambda i: (0, i, 0)),
            pl.BlockSpec((NC, BLK, 16), lambda i: (0, i, 0)),
        ],
        out_shape=[
            out0_shape,
            jax.ShapeDtypeStruct((NC, N, 16), jnp.float32),
            jax.ShapeDtypeStruct((NC, N, 16), jnp.float32),
        ],
    )(acc, hs, ap, dp, b, W, As, Ad)


def _tc4_body(acc_ref, h3_ref, ap_ref, dp_ref, b_ref, bat_ref, out_ref):
    i = pl.program_id(0)
    t = ap_ref[0] + dp_ref[0]
    ws = jnp.exp(jnp.where(t >= 0, t, 0.2 * t))[:, 0:1]
    num = acc_ref[0][:, :64] + acc_ref[1][:, :64] + ws * h3_ref[...]
    den = acc_ref[0][:, 64:65] + acc_ref[1][:, 64:65] + ws
    o = num / den + b_ref[...]
    bf = bat_ref[...]

    @pl.when(i == 0)
    def _():
        out_ref[...] = jnp.full((G, HID), -jnp.inf, jnp.float32)

    bmin = jnp.min(bf)
    bmax = jnp.max(bf)
    for g in range(G):
        @pl.when((bmin <= g) & (g <= bmax))
        def _():
            m = jnp.max(jnp.where(bf == g, o, -jnp.inf), axis=0, keepdims=True)
            out_ref[g:g + 1, :] = jnp.maximum(out_ref[g:g + 1, :], m)

    @pl.when(i == NBLK - 1)
    def _():
        p = out_ref[...]
        out_ref[...] = jnp.where(p > -jnp.float32(3e38), p, 0.0)


def _tc4(acc3, h3, ap3, dp3, b3, batf):
    return pl.pallas_call(
        _tc4_body,
        grid=(NBLK,),
        in_specs=[
            pl.BlockSpec((NC, BLK, 80), lambda i: (0, i, 0)),
            pl.BlockSpec((BLK, HID), lambda i: (i, 0)),
            pl.BlockSpec((NC, BLK, 16), lambda i: (0, i, 0)),
            pl.BlockSpec((NC, BLK, 16), lambda i: (0, i, 0)),
            pl.BlockSpec((1, HID), lambda i: (0, 0)),
            pl.BlockSpec((BLK, 1), lambda i: (i, 0)),
        ],
        out_specs=pl.BlockSpec((G, HID), lambda i: (0, 0)),
        out_shape=jax.ShapeDtypeStruct((G, HID), jnp.float32),
    )(acc3, h3, ap3, dp3, b3, batf)


def _amat(a, heads):
    a2 = a.reshape(heads, HID).astype(jnp.float32)
    eye = jnp.eye(heads, 16, dtype=jnp.float32)
    return (a2[:, :, None] * eye[:, None, :]).reshape(heads * HID, 16)


def kernel(x, edge_index, batch, W1, a_s1, a_d1, b1,
           W2, a_s2, a_d2, b2, W3, a_s3, a_d3, b3):
    As1, Ad1 = _amat(a_s1, H), _amat(a_d1, H)
    As2, Ad2 = _amat(a_s2, H), _amat(a_d2, H)
    As3, Ad3 = _amat(a_s3, 1), _amat(a_d3, 1)

    hs1, ap1, dp1 = _tc1(x, W1, As1, Ad1)
    srce, dste = edge_index[0], edge_index[1]
    acc1 = _sc_edges4(srce, dste, hs1.reshape(NC * N, 128),
                      ap1.reshape(NC * N, 16), dp1.reshape(NC * N, 16))
    hs2, ap2, dp2 = _tc_mid(acc1, hs1, ap1, dp1, b1.reshape(1, -1),
                            W2, As2, Ad2, H)
    acc2 = _sc_edges4(srce, dste, hs2.reshape(NC * N, 128),
                      ap2.reshape(NC * N, 16), dp2.reshape(NC * N, 16))
    h3, ap3, dp3 = _tc_mid(acc2, hs2, ap2, dp2, b2.reshape(1, -1),
                           W3, As3, Ad3, 1)
    acc3 = _sc_edges1(srce, dste, h3,
                      ap3.reshape(NC * N, 16), dp3.reshape(NC * N, 16))
    batf = batch.astype(jnp.float32).reshape(N, 1)
    return _tc4(acc3, h3, ap3, dp3, b3.reshape(1, -1), batf)
